# Initial kernel scaffold; baseline (speedup 1.0000x reference)
import jax
import jax.numpy as jnp
from jax.experimental import pallas as pl


def kernel(queries, keys, values, k, temperature):
    raise NotImplementedError("write your pallas kernel here")


# speedup vs baseline: 19.4541x; 19.4541x over previous
import functools

import jax
import jax.numpy as jnp
from jax import lax
from jax.experimental import pallas as pl
from jax.experimental.pallas import tpu as pltpu
from jax.experimental.pallas import tpu_sc as plsc

Q = 128
D = 64
N = 1000000
VOCAB = 32000
BLK = 8192
NBLK = (N + BLK - 1) // BLK
NPAD = NBLK * BLK
G1 = 128
NG1 = NPAD // G1
NVA = NG1 // 16
BIG = 1e30
BIGF = 3e38
BIGI = 1 << 30


def _tc_body(q_ref, kb_ref, s_ref, m1_ref):
    i = pl.program_id(0)
    q = q_ref[...]
    kb = kb_ref[...]
    dots = lax.dot_general(q, kb, (((1,), (1,)), ((), ())),
                           preferred_element_type=jnp.float32)
    qsq = jnp.sum(q * q, axis=1, keepdims=True)
    ksq = jnp.sum(kb * kb, axis=1)
    s = qsq + ksq[None, :] - 2.0 * dots
    gcol = i * BLK + lax.broadcasted_iota(jnp.int32, (1, BLK), 1)
    s = jnp.where(gcol >= N, BIG, s)
    s_ref[...] = s
    m1_ref[...] = jnp.min(s.reshape(Q, BLK // G1, G1), axis=2)[None]


def _tc_dists(queries, keys):
    return pl.pallas_call(
        _tc_body,
        grid=(NBLK,),
        in_specs=[
            pl.BlockSpec((Q, D), lambda i: (0, 0)),
            pl.BlockSpec((BLK, D), lambda i: (i, 0)),
        ],
        out_specs=[
            pl.BlockSpec((Q, BLK), lambda i: (0, i)),
            pl.BlockSpec((1, Q, BLK // G1), lambda i: (i, 0, 0)),
        ],
        out_shape=[
            jax.ShapeDtypeStruct((Q, NPAD), jnp.float32),
            jax.ShapeDtypeStruct((NBLK, Q, BLK // G1), jnp.float32),
        ],
        compiler_params=pltpu.CompilerParams(
            dimension_semantics=("arbitrary",)),
    )(queries, keys)


def _iota16():
    return lax.iota(jnp.int32, 16)


def _put1(ref, e, val):
    idx = jnp.full((16,), e, dtype=jnp.int32)
    v = jnp.full((16,), val, dtype=ref.dtype)
    plsc.store_scatter(ref, [idx], v, mask=_iota16() == 0)


def _build_mins(buf, mbuf, nv):
    def body(v, c):
        x = buf[pl.ds(v * 16, 16)]
        _put1(mbuf, v, jnp.min(x))
        return c
    lax.fori_loop(0, nv, body, jnp.int32(0))


def _extract64(buf, mbuf, nmv, pos_ref, val_ref):
    iota = _iota16()

    def body(e, c):
        mv = jnp.full((16,), BIGF)
        for j in range(nmv):
            mv = jnp.minimum(mv, mbuf[pl.ds(j * 16, 16)])
        gmin = jnp.min(mv)
        bv = jnp.full((16,), BIGI)
        for j in range(nmv):
            x = mbuf[pl.ds(j * 16, 16)]
            bv = jnp.minimum(bv, jnp.where(x <= gmin, j * 16 + iota, BIGI))
        vstar = jnp.min(bv)
        x = buf[pl.ds(vstar * 16, 16)]
        lane = jnp.min(jnp.where(x <= gmin, iota, BIGI))
        _put1(pos_ref, e, vstar * 16 + lane)
        if val_ref is not None:
            _put1(val_ref, e, gmin)
        x2 = jnp.where(iota == lane, BIGF, x)
        buf[pl.ds(vstar * 16, 16)] = x2
        _put1(mbuf, vstar, jnp.min(x2))
        return c
    lax.fori_loop(0, 64, body, jnp.int32(0))


def _sort64(src, dst):
    iota = _iota16()

    def body(e, c):
        mv = jnp.full((16,), BIGI)
        for j in range(4):
            mv = jnp.minimum(mv, src[pl.ds(j * 16, 16)])
        gm = jnp.min(mv)
        bv = jnp.full((16,), BIGI)
        for j in range(4):
            x = src[pl.ds(j * 16, 16)]
            bv = jnp.minimum(bv, jnp.where(x <= gm, j * 16 + iota, BIGI))
        ps = jnp.min(bv)
        _put1(dst, e, gm)
        _put1(src, ps, BIGI)
        return c
    lax.fori_loop(0, 64, body, jnp.int32(0))


def _sc_body(s2d, m1l, valsv, tvec, out,
             m1row, bbuf, vbuf, mbuf, posb, l1s, rowids, gidxb, dtop, row,
             tv, sem):
    iota = _iota16()
    wid = lax.axis_index("s") * 2 + lax.axis_index("c")

    pltpu.sync_copy(tvec, tv)
    tvv = tv[pl.ds(0, 16)]

    def zbody(v, c):
        row[pl.ds(v * 16, 16)] = jnp.zeros((16,), jnp.float32)
        return c
    lax.fori_loop(0, VOCAB // 16, zbody, jnp.int32(0))

    def qbody(t, c0):
        q = wid * 4 + t

        pltpu.sync_copy(m1l.at[pl.ds(q * NG1, NG1)], m1row)
        _build_mins(m1row, mbuf, NVA)
        xm = mbuf[pl.ds(480, 16)]
        mbuf[pl.ds(480, 16)] = jnp.where(iota < 12, xm, BIGF)
        mbuf[pl.ds(496, 16)] = jnp.full((16,), BIGF)
        _extract64(m1row, mbuf, 32, posb, None)
        _sort64(posb, l1s)

        def fire(e, c):
            lv = l1s[pl.ds((e >> 4) * 16, 16)]
            l1 = jnp.sum(jnp.where(iota == (e & 15), lv, 0))
            pltpu.async_copy(s2d.at[q, pl.ds(l1 * G1, G1)],
                             bbuf.at[pl.ds(e * G1, G1)], sem)
            return c
        lax.fori_loop(0, 64, fire, jnp.int32(0))

        def drain(e, c):
            pltpu.make_async_copy(s2d.at[0, pl.ds(0, G1)],
                                  bbuf.at[pl.ds(0, G1)], sem).wait()
            return c
        lax.fori_loop(0, 64, drain, jnp.int32(0))

        _build_mins(bbuf, mbuf, 512)
        _extract64(bbuf, mbuf, 32, posb, dtop)
        for j in range(4):
            pv = posb[pl.ds(j * 16, 16)]
            gv = plsc.load_gather(l1s, [pv >> 7]) * G1 + (pv & (G1 - 1))
            gidxb[pl.ds(j * 16, 16)] = gv
            rowids[pl.ds(j * 16, 16)] = gv >> 7

        pltpu.async_copy(valsv.at[rowids], vbuf, sem).wait()

        ls = [-dtop[pl.ds(j * 16, 16)] / tvv for j in range(4)]
        m = ls[0]
        for j in range(1, 4):
            m = jnp.maximum(m, ls[j])
        ms = jnp.max(m)
        ws = [jnp.exp(l - ms) for l in ls]
        z = ws[0]
        for j in range(1, 4):
            z = z + ws[j]
        zs = jnp.sum(z)
        for j in range(4):
            gx = gidxb[pl.ds(j * 16, 16)]
            tok = plsc.load_gather(vbuf, [j * 16 + iota, gx & (G1 - 1)])
            plsc.addupdate_scatter(row, [tok], ws[j] / zs)
        pltpu.sync_copy(row, out.at[pl.ds(q * VOCAB, VOCAB)])
        for j in range(4):
            gx = gidxb[pl.ds(j * 16, 16)]
            tok = plsc.load_gather(vbuf, [j * 16 + iota, gx & (G1 - 1)])
            plsc.store_scatter(row, [tok], jnp.zeros((16,), jnp.float32))
        return c0

    lax.fori_loop(0, 4, qbody, jnp.int32(0))


def _sc_select(s2d, m1l, valsv, tvec):
    mesh = plsc.VectorSubcoreMesh(core_axis_name="c", subcore_axis_name="s")
    kern = functools.partial(
        pl.kernel,
        out_type=jax.ShapeDtypeStruct((Q * VOCAB,), jnp.float32),
        mesh=mesh,
        scratch_types=[
            pltpu.VMEM((NG1,), jnp.float32),
            pltpu.VMEM((64 * G1,), jnp.float32),
            pltpu.VMEM((64, G1), jnp.int32),
            pltpu.VMEM((512,), jnp.float32),
            pltpu.VMEM((64,), jnp.int32),
            pltpu.VMEM((64,), jnp.int32),
            pltpu.VMEM((64,), jnp.int32),
            pltpu.VMEM((64,), jnp.int32),
            pltpu.VMEM((64,), jnp.float32),
            pltpu.VMEM((VOCAB,), jnp.float32),
            pltpu.VMEM((16,), jnp.float32),
            pltpu.SemaphoreType.DMA,
        ],
        compiler_params=pltpu.CompilerParams(needs_layout_passes=False),
    )(_sc_body)
    return kern(s2d, m1l, valsv, tvec)


def kernel(queries, keys, values, k, temperature):
    del k
    tvec = jnp.full((16,), temperature, dtype=jnp.float32)
    s, m1 = _tc_dists(queries, keys)
    m1l = m1.transpose(1, 0, 2).reshape(Q * NG1)
    valsv = jnp.pad(values.astype(jnp.int32), (0, NPAD - N)).reshape(NG1, G1)
    out = _sc_select(s, m1l, valsv, tvec)
    return out.reshape(Q, VOCAB)


# speedup vs baseline: 19.6087x; 1.0079x over previous
import functools

import jax
import jax.numpy as jnp
from jax import lax
from jax.experimental import pallas as pl
from jax.experimental.pallas import tpu as pltpu
from jax.experimental.pallas import tpu_sc as plsc

Q = 128
D = 64
N = 1000000
VOCAB = 32000
BLK = 8192
NBLK = (N + BLK - 1) // BLK
NPAD = NBLK * BLK
G1 = 128
NG1 = NPAD // G1
NVA = NG1 // 16
BIG = 1e30
BIGF = 3e38
BIGI = 1 << 30


def _tc_body(q_ref, kb_ref, s_ref, m1_ref):
    i = pl.program_id(0)
    q = q_ref[...]
    kb = kb_ref[...]
    dots = lax.dot_general(q, kb, (((1,), (1,)), ((), ())),
                           preferred_element_type=jnp.float32)
    qsq = jnp.sum(q * q, axis=1, keepdims=True)
    ksq = jnp.sum(kb * kb, axis=1)
    s = qsq + ksq[None, :] - 2.0 * dots
    gcol = i * BLK + lax.broadcasted_iota(jnp.int32, (1, BLK), 1)
    s = jnp.where(gcol >= N, BIG, s)
    s_ref[...] = s[None]
    m1_ref[...] = jnp.min(s.reshape(Q, BLK // G1, G1), axis=2)[None]


def _tc_dists(queries, keys):
    return pl.pallas_call(
        _tc_body,
        grid=(NBLK,),
        in_specs=[
            pl.BlockSpec((Q, D), lambda i: (0, 0)),
            pl.BlockSpec((BLK, D), lambda i: (i, 0)),
        ],
        out_specs=[
            pl.BlockSpec((1, Q, BLK), lambda i: (i, 0, 0)),
            pl.BlockSpec((1, Q, BLK // G1), lambda i: (i, 0, 0)),
        ],
        out_shape=[
            jax.ShapeDtypeStruct((NBLK, Q, BLK), jnp.float32),
            jax.ShapeDtypeStruct((NBLK, Q, BLK // G1), jnp.float32),
        ],
        compiler_params=pltpu.CompilerParams(
            dimension_semantics=("arbitrary",)),
    )(queries, keys)


def _iota16():
    return lax.iota(jnp.int32, 16)


def _put1(ref, e, val):
    idx = jnp.full((16,), e, dtype=jnp.int32)
    v = jnp.full((16,), val, dtype=ref.dtype)
    plsc.store_scatter(ref, [idx], v, mask=_iota16() == 0)


def _build_mins(buf, mbuf, nv):
    def body(v, c):
        x = buf[pl.ds(v * 16, 16)]
        _put1(mbuf, v, jnp.min(x))
        return c
    lax.fori_loop(0, nv, body, jnp.int32(0))


def _extract64(buf, mbuf, nmv, pos_ref, val_ref):
    iota = _iota16()

    def body(e, c):
        mv = jnp.full((16,), BIGF)
        for j in range(nmv):
            mv = jnp.minimum(mv, mbuf[pl.ds(j * 16, 16)])
        gmin = jnp.min(mv)
        bv = jnp.full((16,), BIGI)
        for j in range(nmv):
            x = mbuf[pl.ds(j * 16, 16)]
            bv = jnp.minimum(bv, jnp.where(x <= gmin, j * 16 + iota, BIGI))
        vstar = jnp.min(bv)
        x = buf[pl.ds(vstar * 16, 16)]
        lane = jnp.min(jnp.where(x <= gmin, iota, BIGI))
        _put1(pos_ref, e, vstar * 16 + lane)
        if val_ref is not None:
            _put1(val_ref, e, gmin)
        x2 = jnp.where(iota == lane, BIGF, x)
        buf[pl.ds(vstar * 16, 16)] = x2
        _put1(mbuf, vstar, jnp.min(x2))
        return c
    lax.fori_loop(0, 64, body, jnp.int32(0))


def _sort64(src, dst):
    iota = _iota16()

    def body(e, c):
        mv = jnp.full((16,), BIGI)
        for j in range(4):
            mv = jnp.minimum(mv, src[pl.ds(j * 16, 16)])
        gm = jnp.min(mv)
        bv = jnp.full((16,), BIGI)
        for j in range(4):
            x = src[pl.ds(j * 16, 16)]
            bv = jnp.minimum(bv, jnp.where(x <= gm, j * 16 + iota, BIGI))
        ps = jnp.min(bv)
        _put1(dst, e, gm)
        _put1(src, ps, BIGI)
        return c
    lax.fori_loop(0, 64, body, jnp.int32(0))


def _sc_body(s2d, m1l, valsv, tvec, out,
             m1row, bbuf, vbuf, mbuf, posb, l1s, rowids, gidxb, dtop, row,
             tv, sem):
    iota = _iota16()
    wid = lax.axis_index("s") * 2 + lax.axis_index("c")

    pltpu.sync_copy(tvec, tv)
    tvv = tv[pl.ds(0, 16)]

    def zbody(v, c):
        row[pl.ds(v * 16, 16)] = jnp.zeros((16,), jnp.float32)
        return c
    lax.fori_loop(0, VOCAB // 16, zbody, jnp.int32(0))

    def qbody(t, c0):
        q = wid * 4 + t

        pltpu.sync_copy(m1l.at[pl.ds(q * NG1, NG1)], m1row)
        _build_mins(m1row, mbuf, NVA)
        xm = mbuf[pl.ds(480, 16)]
        mbuf[pl.ds(480, 16)] = jnp.where(iota < 12, xm, BIGF)
        mbuf[pl.ds(496, 16)] = jnp.full((16,), BIGF)
        _extract64(m1row, mbuf, 32, posb, None)
        _sort64(posb, l1s)

        def fire(e, c):
            lv = l1s[pl.ds((e >> 4) * 16, 16)]
            l1 = jnp.sum(jnp.where(iota == (e & 15), lv, 0))
            pltpu.async_copy(
                s2d.at[l1 >> 6, q, pl.ds((l1 & 63) * G1, G1)],
                bbuf.at[pl.ds(e * G1, G1)], sem)
            return c
        lax.fori_loop(0, 64, fire, jnp.int32(0))

        def drain(e, c):
            pltpu.make_async_copy(s2d.at[0, 0, pl.ds(0, G1)],
                                  bbuf.at[pl.ds(0, G1)], sem).wait()
            return c
        lax.fori_loop(0, 64, drain, jnp.int32(0))

        _build_mins(bbuf, mbuf, 512)
        _extract64(bbuf, mbuf, 32, posb, dtop)
        for j in range(4):
            pv = posb[pl.ds(j * 16, 16)]
            gv = plsc.load_gather(l1s, [pv >> 7]) * G1 + (pv & (G1 - 1))
            gidxb[pl.ds(j * 16, 16)] = gv
            rowids[pl.ds(j * 16, 16)] = gv >> 7

        pltpu.async_copy(valsv.at[rowids], vbuf, sem).wait()

        ls = [-dtop[pl.ds(j * 16, 16)] / tvv for j in range(4)]
        m = ls[0]
        for j in range(1, 4):
            m = jnp.maximum(m, ls[j])
        ms = jnp.max(m)
        ws = [jnp.exp(l - ms) for l in ls]
        z = ws[0]
        for j in range(1, 4):
            z = z + ws[j]
        zs = jnp.sum(z)
        for j in range(4):
            gx = gidxb[pl.ds(j * 16, 16)]
            tok = plsc.load_gather(vbuf, [j * 16 + iota, gx & (G1 - 1)])
            plsc.addupdate_scatter(row, [tok], ws[j] / zs)
        pltpu.sync_copy(row, out.at[pl.ds(q * VOCAB, VOCAB)])
        for j in range(4):
            gx = gidxb[pl.ds(j * 16, 16)]
            tok = plsc.load_gather(vbuf, [j * 16 + iota, gx & (G1 - 1)])
            plsc.store_scatter(row, [tok], jnp.zeros((16,), jnp.float32))
        return c0

    lax.fori_loop(0, 4, qbody, jnp.int32(0))


def _sc_select(s2d, m1l, valsv, tvec):
    mesh = plsc.VectorSubcoreMesh(core_axis_name="c", subcore_axis_name="s")
    kern = functools.partial(
        pl.kernel,
        out_type=jax.ShapeDtypeStruct((Q * VOCAB,), jnp.float32),
        mesh=mesh,
        scratch_types=[
            pltpu.VMEM((NG1,), jnp.float32),
            pltpu.VMEM((64 * G1,), jnp.float32),
            pltpu.VMEM((64, G1), jnp.int32),
            pltpu.VMEM((512,), jnp.float32),
            pltpu.VMEM((64,), jnp.int32),
            pltpu.VMEM((64,), jnp.int32),
            pltpu.VMEM((64,), jnp.int32),
            pltpu.VMEM((64,), jnp.int32),
            pltpu.VMEM((64,), jnp.float32),
            pltpu.VMEM((VOCAB,), jnp.float32),
            pltpu.VMEM((16,), jnp.float32),
            pltpu.SemaphoreType.DMA,
        ],
        compiler_params=pltpu.CompilerParams(needs_layout_passes=False),
    )(_sc_body)
    return kern(s2d, m1l, valsv, tvec)


def kernel(queries, keys, values, k, temperature):
    del k
    tvec = jnp.full((16,), temperature, dtype=jnp.float32)
    s, m1 = _tc_dists(queries, keys)
    m1l = m1.transpose(1, 0, 2).reshape(Q * NG1)
    valsv = jnp.pad(values.astype(jnp.int32), (0, NPAD - N)).reshape(NG1, G1)
    out = _sc_select(s, m1l, valsv, tvec)
    return out.reshape(Q, VOCAB)


# speedup vs baseline: 20.9644x; 1.0691x over previous
import functools

import jax
import jax.numpy as jnp
from jax import lax
from jax.experimental import pallas as pl
from jax.experimental.pallas import tpu as pltpu
from jax.experimental.pallas import tpu_sc as plsc

Q = 128
D = 64
N = 1000000
VOCAB = 32000
BLK = 16384
NBLK = (N + BLK - 1) // BLK
NPAD = NBLK * BLK
G1 = 128
GPB = BLK // G1
NG1 = NPAD // G1
NVA = NG1 // 16
BIG = 1e30
BIGF = 3e38
BIGI = 1 << 30


def _tc_body(q_ref, kb_ref, s_ref, m1_ref):
    i = pl.program_id(0)
    q = q_ref[...]
    kb = kb_ref[...]
    dots = lax.dot_general(q, kb, (((1,), (1,)), ((), ())),
                           preferred_element_type=jnp.float32)
    qsq = jnp.sum(q * q, axis=1, keepdims=True)
    ksq = jnp.sum(kb * kb, axis=1)
    s = qsq + ksq[None, :] - 2.0 * dots
    gcol = i * BLK + lax.broadcasted_iota(jnp.int32, (1, BLK), 1)
    s = jnp.where(gcol >= N, BIG, s)
    s_ref[...] = s[None]
    m1_ref[...] = jnp.min(s.reshape(Q, BLK // G1, G1), axis=2)[None]


def _tc_dists(queries, keys):
    return pl.pallas_call(
        _tc_body,
        grid=(NBLK,),
        in_specs=[
            pl.BlockSpec((Q, D), lambda i: (0, 0)),
            pl.BlockSpec((BLK, D), lambda i: (i, 0)),
        ],
        out_specs=[
            pl.BlockSpec((1, Q, BLK), lambda i: (i, 0, 0)),
            pl.BlockSpec((1, Q, BLK // G1), lambda i: (i, 0, 0)),
        ],
        out_shape=[
            jax.ShapeDtypeStruct((NBLK, Q, BLK), jnp.float32),
            jax.ShapeDtypeStruct((NBLK, Q, BLK // G1), jnp.float32),
        ],
        compiler_params=pltpu.CompilerParams(
            dimension_semantics=("arbitrary",)),
    )(queries, keys)


def _iota16():
    return lax.iota(jnp.int32, 16)


def _put1(ref, e, val):
    idx = jnp.full((16,), e, dtype=jnp.int32)
    v = jnp.full((16,), val, dtype=ref.dtype)
    plsc.store_scatter(ref, [idx], v, mask=_iota16() == 0)


def _build_mins(buf, mbuf, nv):
    def body(v, c):
        x = buf[pl.ds(v * 16, 16)]
        _put1(mbuf, v, jnp.min(x))
        return c
    lax.fori_loop(0, nv, body, jnp.int32(0))


def _extract64(buf, mbuf, nmv, pos_ref, val_ref):
    iota = _iota16()

    def body(e, c):
        mv = jnp.full((16,), BIGF)
        for j in range(nmv):
            mv = jnp.minimum(mv, mbuf[pl.ds(j * 16, 16)])
        gmin = jnp.min(mv)
        bv = jnp.full((16,), BIGI)
        for j in range(nmv):
            x = mbuf[pl.ds(j * 16, 16)]
            bv = jnp.minimum(bv, jnp.where(x <= gmin, j * 16 + iota, BIGI))
        vstar = jnp.min(bv)
        x = buf[pl.ds(vstar * 16, 16)]
        lane = jnp.min(jnp.where(x <= gmin, iota, BIGI))
        _put1(pos_ref, e, vstar * 16 + lane)
        if val_ref is not None:
            _put1(val_ref, e, gmin)
        x2 = jnp.where(iota == lane, BIGF, x)
        buf[pl.ds(vstar * 16, 16)] = x2
        _put1(mbuf, vstar, jnp.min(x2))
        return c
    lax.fori_loop(0, 64, body, jnp.int32(0))


def _sort64(src, dst):
    iota = _iota16()

    def body(e, c):
        mv = jnp.full((16,), BIGI)
        for j in range(4):
            mv = jnp.minimum(mv, src[pl.ds(j * 16, 16)])
        gm = jnp.min(mv)
        bv = jnp.full((16,), BIGI)
        for j in range(4):
            x = src[pl.ds(j * 16, 16)]
            bv = jnp.minimum(bv, jnp.where(x <= gm, j * 16 + iota, BIGI))
        ps = jnp.min(bv)
        _put1(dst, e, gm)
        _put1(src, ps, BIGI)
        return c
    lax.fori_loop(0, 64, body, jnp.int32(0))


def _sc_body(s2d, m1l, valsv, tvec, out,
             m1row, bbuf, vbuf, mbuf, posb, l1s, rowids, gidxb, dtop, row,
             tv, sem):
    iota = _iota16()
    wid = lax.axis_index("s") * 2 + lax.axis_index("c")

    pltpu.sync_copy(tvec, tv)
    tvv = tv[pl.ds(0, 16)]

    def zbody(v, c):
        row[pl.ds(v * 16, 16)] = jnp.zeros((16,), jnp.float32)
        return c
    lax.fori_loop(0, VOCAB // 16, zbody, jnp.int32(0))

    def qbody(t, c0):
        q = wid * 4 + t

        pltpu.sync_copy(m1l.at[pl.ds(q * NG1, NG1)], m1row)
        _build_mins(m1row, mbuf, NVA)
        mbuf[pl.ds(496, 16)] = jnp.full((16,), BIGF)
        _extract64(m1row, mbuf, 32, posb, None)
        _sort64(posb, l1s)

        def fire(e, c):
            lv = l1s[pl.ds((e >> 4) * 16, 16)]
            l1 = jnp.sum(jnp.where(iota == (e & 15), lv, 0))
            pltpu.async_copy(
                s2d.at[l1 // GPB, q, pl.ds((l1 % GPB) * G1, G1)],
                bbuf.at[pl.ds(e * G1, G1)], sem)
            return c
        lax.fori_loop(0, 64, fire, jnp.int32(0))

        def drain(e, c):
            pltpu.make_async_copy(s2d.at[0, 0, pl.ds(0, G1)],
                                  bbuf.at[pl.ds(0, G1)], sem).wait()
            return c
        lax.fori_loop(0, 64, drain, jnp.int32(0))

        _build_mins(bbuf, mbuf, 512)
        _extract64(bbuf, mbuf, 32, posb, dtop)
        for j in range(4):
            pv = posb[pl.ds(j * 16, 16)]
            gv = plsc.load_gather(l1s, [pv >> 7]) * G1 + (pv & (G1 - 1))
            gidxb[pl.ds(j * 16, 16)] = gv
            rowids[pl.ds(j * 16, 16)] = gv >> 7

        pltpu.async_copy(valsv.at[rowids], vbuf, sem).wait()

        ls = [-dtop[pl.ds(j * 16, 16)] / tvv for j in range(4)]
        m = ls[0]
        for j in range(1, 4):
            m = jnp.maximum(m, ls[j])
        ms = jnp.max(m)
        ws = [jnp.exp(l - ms) for l in ls]
        z = ws[0]
        for j in range(1, 4):
            z = z + ws[j]
        zs = jnp.sum(z)
        for j in range(4):
            gx = gidxb[pl.ds(j * 16, 16)]
            tok = plsc.load_gather(vbuf, [j * 16 + iota, gx & (G1 - 1)])
            plsc.addupdate_scatter(row, [tok], ws[j] / zs)
        pltpu.sync_copy(row, out.at[q])
        for j in range(4):
            gx = gidxb[pl.ds(j * 16, 16)]
            tok = plsc.load_gather(vbuf, [j * 16 + iota, gx & (G1 - 1)])
            plsc.store_scatter(row, [tok], jnp.zeros((16,), jnp.float32))
        return c0

    lax.fori_loop(0, 4, qbody, jnp.int32(0))


def _sc_select(s2d, m1l, valsv, tvec):
    mesh = plsc.VectorSubcoreMesh(core_axis_name="c", subcore_axis_name="s")
    kern = functools.partial(
        pl.kernel,
        out_type=jax.ShapeDtypeStruct((Q, VOCAB), jnp.float32),
        mesh=mesh,
        scratch_types=[
            pltpu.VMEM((NG1,), jnp.float32),
            pltpu.VMEM((64 * G1,), jnp.float32),
            pltpu.VMEM((64, G1), jnp.int32),
            pltpu.VMEM((512,), jnp.float32),
            pltpu.VMEM((64,), jnp.int32),
            pltpu.VMEM((64,), jnp.int32),
            pltpu.VMEM((64,), jnp.int32),
            pltpu.VMEM((64,), jnp.int32),
            pltpu.VMEM((64,), jnp.float32),
            pltpu.VMEM((VOCAB,), jnp.float32),
            pltpu.VMEM((16,), jnp.float32),
            pltpu.SemaphoreType.DMA,
        ],
        compiler_params=pltpu.CompilerParams(needs_layout_passes=False),
    )(_sc_body)
    return kern(s2d, m1l, valsv, tvec)


def kernel(queries, keys, values, k, temperature):
    del k
    tvec = jnp.full((16,), temperature, dtype=jnp.float32)
    s, m1 = _tc_dists(queries, keys)
    m1l = m1.transpose(1, 0, 2).reshape(Q * NG1)
    valsv = jnp.pad(values.astype(jnp.int32), (0, NPAD - N)).reshape(NG1, G1)
    return _sc_select(s, m1l, valsv, tvec)


# speedup vs baseline: 22.1343x; 1.0558x over previous
import functools

import jax
import jax.numpy as jnp
from jax import lax
from jax.experimental import pallas as pl
from jax.experimental.pallas import tpu as pltpu
from jax.experimental.pallas import tpu_sc as plsc

Q = 128
D = 64
N = 1000000
VOCAB = 32000
BLK = 16384
NBLK = (N + BLK - 1) // BLK
NPAD = NBLK * BLK
G1 = 128
GPB = BLK // G1
NG1 = NPAD // G1
NVA = NG1 // 16
BIG = 1e30
BIGF = 3e38
BIGI = 1 << 30


def _tc_body(q_ref, kb_ref, s_ref, m1_ref):
    i = pl.program_id(0)
    q = q_ref[...]
    kb = kb_ref[...]
    dots = lax.dot_general(q, kb, (((1,), (1,)), ((), ())),
                           preferred_element_type=jnp.float32)
    qsq = jnp.sum(q * q, axis=1, keepdims=True)
    ksq = jnp.sum(kb * kb, axis=1)
    s = qsq + ksq[None, :] - 2.0 * dots
    gcol = i * BLK + lax.broadcasted_iota(jnp.int32, (1, BLK), 1)
    s = jnp.where(gcol >= N, BIG, s)
    s_ref[...] = s[None]
    m1_ref[...] = jnp.min(s.reshape(Q, BLK // G1, G1), axis=2)[None]


def _tc_dists(queries, keys):
    return pl.pallas_call(
        _tc_body,
        grid=(NBLK,),
        in_specs=[
            pl.BlockSpec((Q, D), lambda i: (0, 0)),
            pl.BlockSpec((BLK, D), lambda i: (i, 0)),
        ],
        out_specs=[
            pl.BlockSpec((1, Q, BLK), lambda i: (i, 0, 0)),
            pl.BlockSpec((1, Q, BLK // G1), lambda i: (i, 0, 0)),
        ],
        out_shape=[
            jax.ShapeDtypeStruct((NBLK, Q, BLK), jnp.float32),
            jax.ShapeDtypeStruct((NBLK, Q, BLK // G1), jnp.float32),
        ],
        compiler_params=pltpu.CompilerParams(
            dimension_semantics=("arbitrary",)),
    )(queries, keys)


def _iota16():
    return lax.iota(jnp.int32, 16)


def _put1(ref, e, val):
    idx = jnp.full((16,), e, dtype=jnp.int32)
    v = jnp.full((16,), val, dtype=ref.dtype)
    plsc.store_scatter(ref, [idx], v, mask=_iota16() == 0)


def _build_mins(buf, mbuf, nw):
    iota = _iota16()

    def body(w, c):
        acc = jnp.full((16,), BIGF)
        for j in range(16):
            m = jnp.min(buf[pl.ds((w * 16 + j) * 16, 16)])
            acc = jnp.where(iota == j, m, acc)
        mbuf[pl.ds(w * 16, 16)] = acc
        return c
    lax.fori_loop(0, nw, body, jnp.int32(0))


def _build_mins2(mbuf, mbuf2):
    iota = _iota16()
    for half in range(2):
        acc = jnp.full((16,), BIGF)
        for j in range(16):
            m = jnp.min(mbuf[pl.ds((half * 16 + j) * 16, 16)])
            acc = jnp.where(iota == j, m, acc)
        mbuf2[pl.ds(half * 16, 16)] = acc


def _extract64(buf, mbuf, mbuf2, pos_ref, val_ref):
    iota = _iota16()

    def body(e, c):
        m2a = mbuf2[pl.ds(0, 16)]
        m2b = mbuf2[pl.ds(16, 16)]
        gmin = jnp.min(jnp.minimum(m2a, m2b))
        bw = jnp.minimum(
            jnp.where(m2a <= gmin, iota, BIGI),
            jnp.where(m2b <= gmin, 16 + iota, BIGI))
        wstar = jnp.min(bw)
        mv = mbuf[pl.ds(wstar * 16, 16)]
        mlane = jnp.min(jnp.where(mv <= gmin, iota, BIGI))
        vstar = wstar * 16 + mlane
        x = buf[pl.ds(vstar * 16, 16)]
        lane = jnp.min(jnp.where(x <= gmin, iota, BIGI))
        _put1(pos_ref, e, vstar * 16 + lane)
        if val_ref is not None:
            _put1(val_ref, e, gmin)
        x2 = jnp.where(iota == lane, BIGF, x)
        buf[pl.ds(vstar * 16, 16)] = x2
        mv2 = jnp.where(iota == mlane, jnp.min(x2), mv)
        mbuf[pl.ds(wstar * 16, 16)] = mv2
        _put1(mbuf2, wstar, jnp.min(mv2))
        return c
    lax.fori_loop(0, 64, body, jnp.int32(0))


def _sort64(src, dst):
    iota = _iota16()

    def body(e, c):
        mv = jnp.full((16,), BIGI)
        for j in range(4):
            mv = jnp.minimum(mv, src[pl.ds(j * 16, 16)])
        gm = jnp.min(mv)
        bv = jnp.full((16,), BIGI)
        for j in range(4):
            x = src[pl.ds(j * 16, 16)]
            bv = jnp.minimum(bv, jnp.where(x <= gm, j * 16 + iota, BIGI))
        ps = jnp.min(bv)
        _put1(dst, e, gm)
        _put1(src, ps, BIGI)
        return c
    lax.fori_loop(0, 64, body, jnp.int32(0))


def _sc_body(s2d, m1l, valsv, tvec, out,
             m1row, bbuf, vbuf, mbuf, mbuf2, posb, l1s, rowids, gidxb, dtop,
             row, tv, sem):
    iota = _iota16()
    wid = lax.axis_index("s") * 2 + lax.axis_index("c")

    pltpu.sync_copy(tvec, tv)
    tvv = tv[pl.ds(0, 16)]

    def zbody(v, c):
        for j in range(16):
            row[pl.ds((v * 16 + j) * 16, 16)] = jnp.zeros((16,), jnp.float32)
        return c
    lax.fori_loop(0, VOCAB // 256, zbody, jnp.int32(0))

    def qbody(t, c0):
        q = wid * 4 + t

        pltpu.sync_copy(m1l.at[pl.ds(q * NG1, NG1)], m1row)
        _build_mins(m1row, mbuf, NVA // 16)
        mbuf[pl.ds(496, 16)] = jnp.full((16,), BIGF)
        _build_mins2(mbuf, mbuf2)
        _extract64(m1row, mbuf, mbuf2, posb, None)
        _sort64(posb, l1s)

        def fire(e, c):
            lv = l1s[pl.ds((e >> 4) * 16, 16)]
            l1 = jnp.sum(jnp.where(iota == (e & 15), lv, 0))
            pltpu.async_copy(
                s2d.at[l1 // GPB, q, pl.ds((l1 % GPB) * G1, G1)],
                bbuf.at[pl.ds(e * G1, G1)], sem)
            return c
        lax.fori_loop(0, 64, fire, jnp.int32(0))

        def drain(e, c):
            pltpu.make_async_copy(s2d.at[0, 0, pl.ds(0, G1)],
                                  bbuf.at[pl.ds(0, G1)], sem).wait()
            return c
        lax.fori_loop(0, 64, drain, jnp.int32(0))

        _build_mins(bbuf, mbuf, 32)
        _build_mins2(mbuf, mbuf2)
        _extract64(bbuf, mbuf, mbuf2, posb, dtop)
        for j in range(4):
            pv = posb[pl.ds(j * 16, 16)]
            gv = plsc.load_gather(l1s, [pv >> 7]) * G1 + (pv & (G1 - 1))
            gidxb[pl.ds(j * 16, 16)] = gv
            rowids[pl.ds(j * 16, 16)] = gv >> 7

        pltpu.async_copy(valsv.at[rowids], vbuf, sem).wait()

        ls = [-dtop[pl.ds(j * 16, 16)] / tvv for j in range(4)]
        m = ls[0]
        for j in range(1, 4):
            m = jnp.maximum(m, ls[j])
        ms = jnp.max(m)
        ws = [jnp.exp(l - ms) for l in ls]
        z = ws[0]
        for j in range(1, 4):
            z = z + ws[j]
        zs = jnp.sum(z)
        for j in range(4):
            gx = gidxb[pl.ds(j * 16, 16)]
            tok = plsc.load_gather(vbuf, [j * 16 + iota, gx & (G1 - 1)])
            plsc.addupdate_scatter(row, [tok], ws[j] / zs)
        pltpu.sync_copy(row, out.at[q])
        for j in range(4):
            gx = gidxb[pl.ds(j * 16, 16)]
            tok = plsc.load_gather(vbuf, [j * 16 + iota, gx & (G1 - 1)])
            plsc.store_scatter(row, [tok], jnp.zeros((16,), jnp.float32))
        return c0

    lax.fori_loop(0, 4, qbody, jnp.int32(0))


def _sc_select(s2d, m1l, valsv, tvec):
    mesh = plsc.VectorSubcoreMesh(core_axis_name="c", subcore_axis_name="s")
    kern = functools.partial(
        pl.kernel,
        out_type=jax.ShapeDtypeStruct((Q, VOCAB), jnp.float32),
        mesh=mesh,
        scratch_types=[
            pltpu.VMEM((NG1,), jnp.float32),
            pltpu.VMEM((64 * G1,), jnp.float32),
            pltpu.VMEM((64, G1), jnp.int32),
            pltpu.VMEM((512,), jnp.float32),
            pltpu.VMEM((32,), jnp.float32),
            pltpu.VMEM((64,), jnp.int32),
            pltpu.VMEM((64,), jnp.int32),
            pltpu.VMEM((64,), jnp.int32),
            pltpu.VMEM((64,), jnp.int32),
            pltpu.VMEM((64,), jnp.float32),
            pltpu.VMEM((VOCAB,), jnp.float32),
            pltpu.VMEM((16,), jnp.float32),
            pltpu.SemaphoreType.DMA,
        ],
        compiler_params=pltpu.CompilerParams(needs_layout_passes=False),
    )(_sc_body)
    return kern(s2d, m1l, valsv, tvec)


def kernel(queries, keys, values, k, temperature):
    del k
    tvec = jnp.full((16,), temperature, dtype=jnp.float32)
    s, m1 = _tc_dists(queries, keys)
    m1l = m1.transpose(1, 0, 2).reshape(Q * NG1)
    valsv = jnp.pad(values.astype(jnp.int32), (0, NPAD - N)).reshape(NG1, G1)
    return _sc_select(s, m1l, valsv, tvec)


# speedup vs baseline: 41.7686x; 1.8870x over previous
import functools

import jax
import jax.numpy as jnp
from jax import lax
from jax.experimental import pallas as pl
from jax.experimental.pallas import tpu as pltpu
from jax.experimental.pallas import tpu_sc as plsc

Q = 128
D = 64
N = 1000000
VOCAB = 32000
BLK = 16384
NBLK = (N + BLK - 1) // BLK
NPAD = NBLK * BLK
G1 = 128
GPB = BLK // G1
NG1 = NPAD // G1
NVA = NG1 // 16
BIG = 1e30
BIGF = 3e38
BIGI = 1 << 30


def _tc_body(q_ref, kb_ref, s_ref, m1_ref):
    i = pl.program_id(0)
    q = q_ref[...]
    kb = kb_ref[...]
    dots = lax.dot_general(q, kb, (((1,), (0,)), ((), ())),
                           preferred_element_type=jnp.float32)
    qsq = jnp.sum(q * q, axis=1, keepdims=True)
    ksq = jnp.sum(kb * kb, axis=0)
    s = qsq + ksq[None, :] - 2.0 * dots
    gcol = i * BLK + lax.broadcasted_iota(jnp.int32, (1, BLK), 1)
    s = jnp.where(gcol >= N, BIG, s)
    s_ref[...] = s[None]
    m1_ref[...] = jnp.min(s.reshape(Q, BLK // G1, G1), axis=2)[None]


def _tc_dists(queries, keys):
    return pl.pallas_call(
        _tc_body,
        grid=(NBLK,),
        in_specs=[
            pl.BlockSpec((Q, D), lambda i: (0, 0)),
            pl.BlockSpec((D, BLK), lambda i: (0, i)),
        ],
        out_specs=[
            pl.BlockSpec((1, Q, BLK), lambda i: (i, 0, 0)),
            pl.BlockSpec((1, Q, BLK // G1), lambda i: (i, 0, 0)),
        ],
        out_shape=[
            jax.ShapeDtypeStruct((NBLK, Q, BLK), jnp.float32),
            jax.ShapeDtypeStruct((NBLK, Q, BLK // G1), jnp.float32),
        ],
        compiler_params=pltpu.CompilerParams(
            dimension_semantics=("arbitrary",)),
    )(queries, keys)


def _iota16():
    return lax.iota(jnp.int32, 16)


def _put1(ref, e, val):
    idx = jnp.full((16,), e, dtype=jnp.int32)
    v = jnp.full((16,), val, dtype=ref.dtype)
    plsc.store_scatter(ref, [idx], v, mask=_iota16() == 0)


def _build_mins(buf, mbuf, nw):
    iota = _iota16()

    def body(w, c):
        acc = jnp.full((16,), BIGF)
        for j in range(16):
            m = jnp.min(buf[pl.ds((w * 16 + j) * 16, 16)])
            acc = jnp.where(iota == j, m, acc)
        mbuf[pl.ds(w * 16, 16)] = acc
        return c
    lax.fori_loop(0, nw, body, jnp.int32(0))


def _build_mins2(mbuf, mbuf2):
    iota = _iota16()
    for half in range(2):
        acc = jnp.full((16,), BIGF)
        for j in range(16):
            m = jnp.min(mbuf[pl.ds((half * 16 + j) * 16, 16)])
            acc = jnp.where(iota == j, m, acc)
        mbuf2[pl.ds(half * 16, 16)] = acc


def _extract64(buf, mbuf, mbuf2, pos_ref, val_ref):
    iota = _iota16()

    def body(e, c):
        m2a = mbuf2[pl.ds(0, 16)]
        m2b = mbuf2[pl.ds(16, 16)]
        gmin = jnp.min(jnp.minimum(m2a, m2b))
        bw = jnp.minimum(
            jnp.where(m2a <= gmin, iota, BIGI),
            jnp.where(m2b <= gmin, 16 + iota, BIGI))
        wstar = jnp.min(bw)
        mv = mbuf[pl.ds(wstar * 16, 16)]
        mlane = jnp.min(jnp.where(mv <= gmin, iota, BIGI))
        vstar = wstar * 16 + mlane
        x = buf[pl.ds(vstar * 16, 16)]
        lane = jnp.min(jnp.where(x <= gmin, iota, BIGI))
        _put1(pos_ref, e, vstar * 16 + lane)
        if val_ref is not None:
            _put1(val_ref, e, gmin)
        x2 = jnp.where(iota == lane, BIGF, x)
        buf[pl.ds(vstar * 16, 16)] = x2
        mv2 = jnp.where(iota == mlane, jnp.min(x2), mv)
        mbuf[pl.ds(wstar * 16, 16)] = mv2
        _put1(mbuf2, wstar, jnp.min(mv2))
        return c
    lax.fori_loop(0, 64, body, jnp.int32(0))


def _sort64(src, dst):
    iota = _iota16()

    def body(e, c):
        mv = jnp.full((16,), BIGI)
        for j in range(4):
            mv = jnp.minimum(mv, src[pl.ds(j * 16, 16)])
        gm = jnp.min(mv)
        bv = jnp.full((16,), BIGI)
        for j in range(4):
            x = src[pl.ds(j * 16, 16)]
            bv = jnp.minimum(bv, jnp.where(x <= gm, j * 16 + iota, BIGI))
        ps = jnp.min(bv)
        _put1(dst, e, gm)
        _put1(src, ps, BIGI)
        return c
    lax.fori_loop(0, 64, body, jnp.int32(0))


def _sc_body(s2d, m1l, valsv, tvec, out,
             m1row, bbuf, vbuf, mbuf, mbuf2, posb, l1s, rowids, gidxb, dtop,
             row, tv, sem):
    iota = _iota16()
    wid = lax.axis_index("s") * 2 + lax.axis_index("c")

    pltpu.sync_copy(tvec, tv)
    tvv = tv[pl.ds(0, 16)]

    def zbody(v, c):
        for j in range(16):
            row[pl.ds((v * 16 + j) * 16, 16)] = jnp.zeros((16,), jnp.float32)
        return c
    lax.fori_loop(0, VOCAB // 256, zbody, jnp.int32(0))

    def qbody(t, c0):
        q = wid * 4 + t

        pltpu.sync_copy(m1l.at[pl.ds(q * NG1, NG1)], m1row)
        _build_mins(m1row, mbuf, NVA // 16)
        mbuf[pl.ds(496, 16)] = jnp.full((16,), BIGF)
        _build_mins2(mbuf, mbuf2)
        _extract64(m1row, mbuf, mbuf2, posb, None)
        _sort64(posb, l1s)

        def fire(e, c):
            lv = l1s[pl.ds((e >> 4) * 16, 16)]
            l1 = jnp.sum(jnp.where(iota == (e & 15), lv, 0))
            pltpu.async_copy(
                s2d.at[l1 // GPB, q, pl.ds((l1 % GPB) * G1, G1)],
                bbuf.at[pl.ds(e * G1, G1)], sem)
            return c
        lax.fori_loop(0, 64, fire, jnp.int32(0))

        def drain(e, c):
            pltpu.make_async_copy(s2d.at[0, 0, pl.ds(0, G1)],
                                  bbuf.at[pl.ds(0, G1)], sem).wait()
            return c
        lax.fori_loop(0, 64, drain, jnp.int32(0))

        _build_mins(bbuf, mbuf, 32)
        _build_mins2(mbuf, mbuf2)
        _extract64(bbuf, mbuf, mbuf2, posb, dtop)
        for j in range(4):
            pv = posb[pl.ds(j * 16, 16)]
            gv = plsc.load_gather(l1s, [pv >> 7]) * G1 + (pv & (G1 - 1))
            gidxb[pl.ds(j * 16, 16)] = gv
            rowids[pl.ds(j * 16, 16)] = gv >> 7

        pltpu.async_copy(valsv.at[rowids], vbuf, sem).wait()

        ls = [-dtop[pl.ds(j * 16, 16)] / tvv for j in range(4)]
        m = ls[0]
        for j in range(1, 4):
            m = jnp.maximum(m, ls[j])
        ms = jnp.max(m)
        ws = [jnp.exp(l - ms) for l in ls]
        z = ws[0]
        for j in range(1, 4):
            z = z + ws[j]
        zs = jnp.sum(z)
        for j in range(4):
            gx = gidxb[pl.ds(j * 16, 16)]
            tok = plsc.load_gather(vbuf, [j * 16 + iota, gx & (G1 - 1)])
            plsc.addupdate_scatter(row, [tok], ws[j] / zs)
        pltpu.sync_copy(row, out.at[q])
        for j in range(4):
            gx = gidxb[pl.ds(j * 16, 16)]
            tok = plsc.load_gather(vbuf, [j * 16 + iota, gx & (G1 - 1)])
            plsc.store_scatter(row, [tok], jnp.zeros((16,), jnp.float32))
        return c0

    lax.fori_loop(0, 4, qbody, jnp.int32(0))


def _sc_select(s2d, m1l, valsv, tvec):
    mesh = plsc.VectorSubcoreMesh(core_axis_name="c", subcore_axis_name="s")
    kern = functools.partial(
        pl.kernel,
        out_type=jax.ShapeDtypeStruct((Q, VOCAB), jnp.float32),
        mesh=mesh,
        scratch_types=[
            pltpu.VMEM((NG1,), jnp.float32),
            pltpu.VMEM((64 * G1,), jnp.float32),
            pltpu.VMEM((64, G1), jnp.int32),
            pltpu.VMEM((512,), jnp.float32),
            pltpu.VMEM((32,), jnp.float32),
            pltpu.VMEM((64,), jnp.int32),
            pltpu.VMEM((64,), jnp.int32),
            pltpu.VMEM((64,), jnp.int32),
            pltpu.VMEM((64,), jnp.int32),
            pltpu.VMEM((64,), jnp.float32),
            pltpu.VMEM((VOCAB,), jnp.float32),
            pltpu.VMEM((16,), jnp.float32),
            pltpu.SemaphoreType.DMA,
        ],
        compiler_params=pltpu.CompilerParams(needs_layout_passes=False),
    )(_sc_body)
    return kern(s2d, m1l, valsv, tvec)


def kernel(queries, keys, values, k, temperature):
    del k
    tvec = jnp.full((16,), temperature, dtype=jnp.float32)
    s, m1 = _tc_dists(queries, keys.T)
    m1l = m1.transpose(1, 0, 2).reshape(Q * NG1)
    valsv = jnp.pad(values.astype(jnp.int32), (0, NPAD - N)).reshape(NG1, G1)
    return _sc_select(s, m1l, valsv, tvec)


# speedup vs baseline: 42.5306x; 1.0182x over previous
import functools

import jax
import jax.numpy as jnp
from jax import lax
from jax.experimental import pallas as pl
from jax.experimental.pallas import tpu as pltpu
from jax.experimental.pallas import tpu_sc as plsc

Q = 128
D = 64
N = 1000000
VOCAB = 32000
BLK = 16384
NBLK = (N + BLK - 1) // BLK
NPAD = NBLK * BLK
G1 = 128
GPB = BLK // G1
NG1 = NPAD // G1
NVA = NG1 // 16
BIG = 1e30
BIGF = 3e38
BIGI = 1 << 30


def _tc_body(q_ref, kb_ref, s_ref, m1_ref):
    i = pl.program_id(0)
    q = q_ref[...]
    kb = kb_ref[...]
    dots = lax.dot_general(q, kb, (((1,), (0,)), ((), ())),
                           preferred_element_type=jnp.float32)
    qsq = jnp.sum(q * q, axis=1, keepdims=True)
    ksq = jnp.sum(kb * kb, axis=0)
    s = qsq + ksq[None, :] - 2.0 * dots
    gcol = i * BLK + lax.broadcasted_iota(jnp.int32, (1, BLK), 1)
    s = jnp.where(gcol >= N, BIG, s)
    s_ref[...] = s[None]
    m1_ref[...] = jnp.min(s.reshape(Q, BLK // G1, G1), axis=2)[None]


def _tc_dists(queries, keys):
    return pl.pallas_call(
        _tc_body,
        grid=(NBLK,),
        in_specs=[
            pl.BlockSpec((Q, D), lambda i: (0, 0)),
            pl.BlockSpec((D, BLK), lambda i: (0, i)),
        ],
        out_specs=[
            pl.BlockSpec((1, Q, BLK), lambda i: (i, 0, 0)),
            pl.BlockSpec((1, Q, BLK // G1), lambda i: (i, 0, 0)),
        ],
        out_shape=[
            jax.ShapeDtypeStruct((NBLK, Q, BLK), jnp.float32),
            jax.ShapeDtypeStruct((NBLK, Q, BLK // G1), jnp.float32),
        ],
        compiler_params=pltpu.CompilerParams(
            dimension_semantics=("arbitrary",)),
    )(queries, keys)


def _iota16():
    return lax.iota(jnp.int32, 16)


def _put1(ref, e, val):
    idx = jnp.full((16,), e, dtype=jnp.int32)
    v = jnp.full((16,), val, dtype=ref.dtype)
    plsc.store_scatter(ref, [idx], v, mask=_iota16() == 0)


def _build_mins(buf, mbuf, nw):
    iota = _iota16()

    def body(w, c):
        acc = jnp.full((16,), BIGF)
        for j in range(16):
            m = jnp.min(buf[pl.ds((w * 16 + j) * 16, 16)])
            acc = jnp.where(iota == j, m, acc)
        mbuf[pl.ds(w * 16, 16)] = acc
        return c
    lax.fori_loop(0, nw, body, jnp.int32(0))


def _build_mins2(mbuf, mbuf2):
    iota = _iota16()
    for half in range(2):
        acc = jnp.full((16,), BIGF)
        for j in range(16):
            m = jnp.min(mbuf[pl.ds((half * 16 + j) * 16, 16)])
            acc = jnp.where(iota == j, m, acc)
        mbuf2[pl.ds(half * 16, 16)] = acc


def _extract64(buf, mbuf, mbuf2, pos_ref, val_ref):
    iota = _iota16()

    def body(e, c):
        m2a = mbuf2[pl.ds(0, 16)]
        m2b = mbuf2[pl.ds(16, 16)]
        gmin = jnp.min(jnp.minimum(m2a, m2b))
        bw = jnp.minimum(
            jnp.where(m2a <= gmin, iota, BIGI),
            jnp.where(m2b <= gmin, 16 + iota, BIGI))
        wstar = jnp.min(bw)
        mv = mbuf[pl.ds(wstar * 16, 16)]
        mlane = jnp.min(jnp.where(mv <= gmin, iota, BIGI))
        vstar = wstar * 16 + mlane
        x = buf[pl.ds(vstar * 16, 16)]
        lane = jnp.min(jnp.where(x <= gmin, iota, BIGI))
        _put1(pos_ref, e, vstar * 16 + lane)
        if val_ref is not None:
            _put1(val_ref, e, gmin)
        x2 = jnp.where(iota == lane, BIGF, x)
        buf[pl.ds(vstar * 16, 16)] = x2
        mv2 = jnp.where(iota == mlane, jnp.min(x2), mv)
        mbuf[pl.ds(wstar * 16, 16)] = mv2
        _put1(mbuf2, wstar, jnp.min(mv2))
        return c
    lax.fori_loop(0, 64, body, jnp.int32(0))


def _sort64(src, dst):
    iota = _iota16()

    def body(e, c):
        mv = jnp.full((16,), BIGI)
        for j in range(4):
            mv = jnp.minimum(mv, src[pl.ds(j * 16, 16)])
        gm = jnp.min(mv)
        bv = jnp.full((16,), BIGI)
        for j in range(4):
            x = src[pl.ds(j * 16, 16)]
            bv = jnp.minimum(bv, jnp.where(x <= gm, j * 16 + iota, BIGI))
        ps = jnp.min(bv)
        _put1(dst, e, gm)
        _put1(src, ps, BIGI)
        return c
    lax.fori_loop(0, 64, body, jnp.int32(0))


def _sc_body(s2d, m1l, valsv, tvec, out,
             m1row, bbuf, vbuf, mbuf, mbuf2, posb, l1s, rowids, gidxb, dtop,
             prevtok, row, tv, sem, sem2, sem3):
    iota = _iota16()
    wid = lax.axis_index("s") * 2 + lax.axis_index("c")

    pltpu.sync_copy(tvec, tv)
    tvv = tv[pl.ds(0, 16)]

    def zbody(v, c):
        for j in range(16):
            row[pl.ds((v * 16 + j) * 16, 16)] = jnp.zeros((16,), jnp.float32)
        return c
    lax.fori_loop(0, VOCAB // 256, zbody, jnp.int32(0))

    pltpu.async_copy(m1l.at[pl.ds(wid * 4 * NG1, NG1)], m1row, sem2)

    def qbody(t, c0):
        q = wid * 4 + t

        pltpu.make_async_copy(m1l.at[pl.ds(0, NG1)], m1row, sem2).wait()
        _build_mins(m1row, mbuf, NVA // 16)
        mbuf[pl.ds(496, 16)] = jnp.full((16,), BIGF)
        _build_mins2(mbuf, mbuf2)
        _extract64(m1row, mbuf, mbuf2, posb, None)
        _sort64(posb, l1s)

        @pl.when(t < 3)
        def _():
            pltpu.async_copy(m1l.at[pl.ds((q + 1) * NG1, NG1)], m1row, sem2)

        def fire(e, c):
            lv = l1s[pl.ds((e >> 4) * 16, 16)]
            l1 = jnp.sum(jnp.where(iota == (e & 15), lv, 0))
            pltpu.async_copy(
                s2d.at[l1 // GPB, q, pl.ds((l1 % GPB) * G1, G1)],
                bbuf.at[pl.ds(e * G1, G1)], sem)
            return c
        lax.fori_loop(0, 64, fire, jnp.int32(0))

        def drain(e, c):
            pltpu.make_async_copy(s2d.at[0, 0, pl.ds(0, G1)],
                                  bbuf.at[pl.ds(0, G1)], sem).wait()
            return c
        lax.fori_loop(0, 64, drain, jnp.int32(0))

        _build_mins(bbuf, mbuf, 32)
        _build_mins2(mbuf, mbuf2)
        _extract64(bbuf, mbuf, mbuf2, posb, dtop)
        for j in range(4):
            pv = posb[pl.ds(j * 16, 16)]
            gv = plsc.load_gather(l1s, [pv >> 7]) * G1 + (pv & (G1 - 1))
            gidxb[pl.ds(j * 16, 16)] = gv
            rowids[pl.ds(j * 16, 16)] = gv >> 7

        pltpu.async_copy(valsv.at[rowids], vbuf, sem).wait()

        ls = [-dtop[pl.ds(j * 16, 16)] / tvv for j in range(4)]
        m = ls[0]
        for j in range(1, 4):
            m = jnp.maximum(m, ls[j])
        ms = jnp.max(m)
        ws = [jnp.exp(l - ms) for l in ls]
        z = ws[0]
        for j in range(1, 4):
            z = z + ws[j]
        zs = jnp.sum(z)

        @pl.when(t > 0)
        def _():
            pltpu.make_async_copy(row, out.at[0], sem3).wait()
            for j in range(4):
                pt = prevtok[pl.ds(j * 16, 16)]
                plsc.store_scatter(row, [pt], jnp.zeros((16,), jnp.float32))

        for j in range(4):
            gx = gidxb[pl.ds(j * 16, 16)]
            tok = plsc.load_gather(vbuf, [j * 16 + iota, gx & (G1 - 1)])
            plsc.addupdate_scatter(row, [tok], ws[j] / zs)
            prevtok[pl.ds(j * 16, 16)] = tok
        pltpu.async_copy(row, out.at[q], sem3)
        return c0

    lax.fori_loop(0, 4, qbody, jnp.int32(0))
    pltpu.make_async_copy(row, out.at[0], sem3).wait()


def _sc_select(s2d, m1l, valsv, tvec):
    mesh = plsc.VectorSubcoreMesh(core_axis_name="c", subcore_axis_name="s")
    kern = functools.partial(
        pl.kernel,
        out_type=jax.ShapeDtypeStruct((Q, VOCAB), jnp.float32),
        mesh=mesh,
        scratch_types=[
            pltpu.VMEM((NG1,), jnp.float32),
            pltpu.VMEM((64 * G1,), jnp.float32),
            pltpu.VMEM((64, G1), jnp.int32),
            pltpu.VMEM((512,), jnp.float32),
            pltpu.VMEM((32,), jnp.float32),
            pltpu.VMEM((64,), jnp.int32),
            pltpu.VMEM((64,), jnp.int32),
            pltpu.VMEM((64,), jnp.int32),
            pltpu.VMEM((64,), jnp.int32),
            pltpu.VMEM((64,), jnp.float32),
            pltpu.VMEM((64,), jnp.int32),
            pltpu.VMEM((VOCAB,), jnp.float32),
            pltpu.VMEM((16,), jnp.float32),
            pltpu.SemaphoreType.DMA,
            pltpu.SemaphoreType.DMA,
            pltpu.SemaphoreType.DMA,
        ],
        compiler_params=pltpu.CompilerParams(needs_layout_passes=False),
    )(_sc_body)
    return kern(s2d, m1l, valsv, tvec)


def kernel(queries, keys, values, k, temperature):
    del k
    tvec = jnp.full((16,), temperature, dtype=jnp.float32)
    s, m1 = _tc_dists(queries, keys.T)
    m1l = m1.transpose(1, 0, 2).reshape(Q * NG1)
    valsv = jnp.pad(values.astype(jnp.int32), (0, NPAD - N)).reshape(NG1, G1)
    return _sc_select(s, m1l, valsv, tvec)
